# baseline (speedup 1.0000x reference)
import functools

import jax
import jax.numpy as jnp
from jax import lax
from jax.experimental import pallas as pl
from jax.experimental.pallas import tpu as pltpu
from jax.experimental.pallas import tpu_sc as plsc

N = 10000
F = 11
D = 128
H = 4
C = 32
L = 4

NP = 10016
RPS = NP // 16
W = 144
CH = 56
NSUB = 32


def _ceil_to(x, m):
    return (x + m - 1) // m * m


def _rowmask():
    rows = lax.broadcasted_iota(jnp.int32, (NP, 1), 0)
    return rows < N


def _embed_body(nf_ref, w_ref, b_ref, x_ref):
    x = jnp.dot(nf_ref[...], w_ref[...], preferred_element_type=jnp.float32)
    x = jnp.maximum(x + b_ref[...], 0.0)
    x_ref[...] = jnp.where(_rowmask(), x, 0.0)


def _tc_embed(nf_pad, W_emb_pad, b_emb):
    return pl.pallas_call(
        _embed_body,
        out_shape=jax.ShapeDtypeStruct((NP, D), jnp.float32),
    )(nf_pad, W_emb_pad, b_emb)


def _pre_body(x_ref, wl_ref, bl_ref, wr_ref, br_ref, a_ref, xl_ref, xre_ref):
    x = x_ref[...]
    mask = _rowmask()
    xl = jnp.dot(x, wl_ref[...], preferred_element_type=jnp.float32) + bl_ref[...]
    xr = jnp.dot(x, wr_ref[...], preferred_element_type=jnp.float32) + br_ref[...]
    xl = jnp.where(mask, xl, 0.0)
    xr = jnp.where(mask, xr, 0.0)
    s = xl + xr
    m = jnp.maximum(s, 0.2 * s)
    c = jnp.dot(m, a_ref[...], preferred_element_type=jnp.float32)
    xl_ref[...] = xl
    xre_ref[...] = jnp.concatenate(
        [xr, c, jnp.zeros((NP, W - D - H), jnp.float32)], axis=1)


def _tc_pre(x, Wl_i, bl_i, Wr_i, br_i, A_i):
    return pl.pallas_call(
        _pre_body,
        out_shape=(
            jax.ShapeDtypeStruct((NP, D), jnp.float32),
            jax.ShapeDtypeStruct((NP, W), jnp.float32),
        ),
    )(x, Wl_i, bl_i, Wr_i, br_i, A_i)


def _post_x(u_ref, bias_ref, res_ref):
    u = u_ref[0] + u_ref[1]
    den = jnp.concatenate(
        [jnp.broadcast_to(u[:, D + h:D + h + 1], (NP, C)) for h in range(H)],
        axis=1)
    out = u[:, :D] / (den + 1e-16)
    hv = jnp.maximum(out + bias_ref[...], 0.0)
    return jnp.where(_rowmask(), hv + res_ref[...], 0.0)


def _post_body(u_ref, bias_ref, res_ref, x_ref):
    x_ref[...] = _post_x(u_ref, bias_ref, res_ref)


def _tc_post(U, bias_i, res):
    return pl.pallas_call(
        _post_body,
        out_shape=jax.ShapeDtypeStruct((NP, D), jnp.float32),
    )(U, bias_i, res)


def _post_final_body(u_ref, bias_ref, res_ref, x_ref, ge_ref):
    x = _post_x(u_ref, bias_ref, res_ref)
    x_ref[...] = x
    gmean = jnp.sum(x, axis=0) * (1.0 / N)
    gmax = jnp.max(x, axis=0)
    ge_ref[...] = jnp.concatenate([gmean[None, :], gmax[None, :]], axis=0)


def _tc_post_final(U, bias_i, res):
    return pl.pallas_call(
        _post_final_body,
        out_shape=(
            jax.ShapeDtypeStruct((NP, D), jnp.float32),
            jax.ShapeDtypeStruct((2, D), jnp.float32),
        ),
    )(U, bias_i, res)


@functools.lru_cache(maxsize=None)
def _make_edge_pass(epad):
    k_chunks = epad // (NSUB * CH)
    assert k_chunks % 2 == 0
    mesh = plsc.VectorSubcoreMesh(core_axis_name="c", subcore_axis_name="s")

    nbuf = 2
    scratch = []
    for _ in range(nbuf):
        scratch += [
            pltpu.VMEM((CH,), jnp.int32),
            pltpu.VMEM((CH,), jnp.int32),
            pltpu.VMEM((CH,), jnp.int32),
            pltpu.VMEM((CH, D), jnp.float32),
            pltpu.VMEM((CH, W), jnp.float32),
            pltpu.SemaphoreType.DMA,
            pltpu.SemaphoreType.DMA,
        ]
    scratch += [
        pltpu.VMEM((CH, W), jnp.float32),
        pltpu.VMEM((D,), jnp.float32),
        pltpu.VMEM_SHARED((NP, W), jnp.float32),
    ]

    @functools.partial(
        pl.kernel,
        mesh=mesh,
        compiler_params=pltpu.CompilerParams(
            use_tc_tiling_on_sc=False, needs_layout_passes=False),
        out_type=jax.ShapeDtypeStruct((2, NP, W), jnp.float32),
        scratch_types=scratch,
    )
    def edge_kernel(xl_hbm, xre_hbm, src_hbm, dst_hbm, att_hbm, out_hbm,
                    *refs):
        bufs = []
        for b in range(nbuf):
            sidx, didx, didxs, xlv, xrv, semi, semg = refs[7 * b:7 * b + 7]
            bufs.append(dict(sidx=sidx, didx=didx, didxs=didxs, xlv=xlv,
                             xrv=xrv, semi=semi, semg=semg))
        msgv, attv, usp = refs[7 * nbuf:]

        cid = lax.axis_index("c")
        sid = lax.axis_index("s")
        wid = sid * 2 + cid
        cbase = wid * k_chunks

        def _zrow(e, carry):
            for j in range(W // 16):
                msgv[e, pl.ds(16 * j, 16)] = jnp.zeros((16,), jnp.float32)
            return carry

        lax.fori_loop(0, CH, _zrow, 0)
        row0 = sid * RPS
        nfull = RPS // CH
        for t in range(nfull):
            pltpu.sync_copy(msgv, usp.at[pl.ds(row0 + t * CH, CH)])
        rem = RPS - nfull * CH
        if rem:
            pltpu.sync_copy(msgv.at[pl.ds(0, rem)],
                            usp.at[pl.ds(row0 + nfull * CH, rem)])
        pltpu.sync_copy(att_hbm, attv)
        plsc.subcore_barrier()

        atts = [attv[pl.ds(16 * j, 16)] for j in range(D // 16)]
        iot = jnp.arange(16, dtype=jnp.int32)

        def _fetch_idx(g, buf):
            base = (cbase + g) * CH
            pltpu.async_copy(src_hbm.at[pl.ds(base, CH)], buf["sidx"],
                             buf["semi"])
            pltpu.async_copy(dst_hbm.at[pl.ds(base, CH)], buf["didx"],
                             buf["semi"])

        def _wait_idx(buf):
            pltpu.make_async_copy(src_hbm.at[pl.ds(0, CH)], buf["sidx"],
                                  buf["semi"]).wait()
            pltpu.make_async_copy(dst_hbm.at[pl.ds(0, CH)], buf["didx"],
                                  buf["semi"]).wait()

        def _start_gathers(buf):
            pltpu.async_copy(xl_hbm.at[buf["sidx"]], buf["xlv"], buf["semg"])
            pltpu.async_copy(xre_hbm.at[buf["didx"]], buf["xrv"], buf["semg"])
            for j in range(0, CH, 16):
                j0 = min(j, CH - 16)
                buf["didxs"][pl.ds(j0, 16)] = buf["didx"][pl.ds(j0, 16)]

        def _wait_gathers(buf):
            pltpu.make_async_copy(xl_hbm.at[buf["sidx"]], buf["xlv"],
                                  buf["semg"]).wait()
            pltpu.make_async_copy(xre_hbm.at[buf["didx"]], buf["xrv"],
                                  buf["semg"]).wait()

        def _compute_scatter(buf):
            xlv, xrv = buf["xlv"], buf["xrv"]

            @plsc.parallel_loop(0, CH, unroll=4)
            def _edge(e):
                xls = [xlv[e, pl.ds(16 * j, 16)] for j in range(D // 16)]
                cv = xrv[e, pl.ds(D, 16)]
                dvals = []
                for h in range(H):
                    acc = None
                    for jj in (2 * h, 2 * h + 1):
                        s = xls[jj] + xrv[e, pl.ds(16 * jj, 16)]
                        m = jnp.maximum(s, 0.2 * s)
                        am = m * atts[jj]
                        acc = am if acc is None else acc + am
                    dvals.append(jnp.sum(acc) - cv[h])
                tail = jnp.full((16,), 0.0, jnp.float32)
                for h in range(H):
                    bex = jnp.exp(jnp.full((16,), dvals[h], jnp.float32))
                    msgv[e, pl.ds(32 * h, 16)] = xls[2 * h] * bex
                    msgv[e, pl.ds(32 * h + 16, 16)] = xls[2 * h + 1] * bex
                    tail = jnp.where(iot == h, bex, tail)
                msgv[e, pl.ds(D, 16)] = tail

            pltpu.sync_copy(msgv, usp.at[buf["didxs"]], add=True)

        pltpu.sync_copy(src_hbm.at[pl.ds(cbase * CH, CH)], bufs[0]["sidx"])
        pltpu.sync_copy(dst_hbm.at[pl.ds(cbase * CH, CH)], bufs[0]["didx"])
        _start_gathers(bufs[0])
        _fetch_idx(1, bufs[1])

        def _halfstep(g, cur, nxt):
            @pl.when(g + 1 < k_chunks)
            def _():
                _wait_idx(nxt)
                _start_gathers(nxt)

            _wait_gathers(cur)

            @pl.when(g + 2 < k_chunks)
            def _():
                _fetch_idx(g + 2, cur)

            _compute_scatter(cur)

        def _pair(t, carry):
            _halfstep(2 * t, bufs[0], bufs[1])
            _halfstep(2 * t + 1, bufs[1], bufs[0])
            return carry

        lax.fori_loop(0, k_chunks // 2, _pair, 0)
        plsc.subcore_barrier()

        for t in range(nfull):
            pltpu.sync_copy(usp.at[pl.ds(row0 + t * CH, CH)],
                            out_hbm.at[cid, pl.ds(row0 + t * CH, CH)])
        pltpu.sync_copy(usp.at[pl.ds(row0 + nfull * CH, rem)],
                        out_hbm.at[cid, pl.ds(row0 + nfull * CH, rem)])

    return edge_kernel


def kernel(node_features, edge_index, num_nodes, W_emb, b_emb,
           Wl, bl, Wr, br, att, bias):
    f32 = jnp.float32
    e_real = edge_index.shape[1] + N
    epad = _ceil_to(e_real, 2 * NSUB * CH)

    nf_pad = jnp.zeros((NP, 16), f32).at[:N, :F].set(node_features)
    wemb_pad = jnp.zeros((16, D), f32).at[:F, :].set(W_emb)
    loop = jnp.arange(N, dtype=jnp.int32)
    padfill = jnp.full((epad - e_real,), N, jnp.int32)
    src = jnp.concatenate([edge_index[0], loop, padfill])
    dst = jnp.concatenate([edge_index[1], loop, padfill])
    attf = att.reshape(L, H * C)
    onehot = jnp.repeat(jnp.eye(H, dtype=f32), C, axis=0)

    edge_pass = _make_edge_pass(epad)

    x = _tc_embed(nf_pad, wemb_pad, b_emb.reshape(1, D))
    ge = None
    for i in range(L):
        a_mat = attf[i][:, None] * onehot
        xl_tab, xre_tab = _tc_pre(x, Wl[i], bl[i].reshape(1, D),
                                  Wr[i], br[i].reshape(1, D), a_mat)
        u_acc = edge_pass(xl_tab, xre_tab, src, dst, attf[i])
        res = jnp.zeros((NP, D), f32) if i == 0 else x
        if i < L - 1:
            x = _tc_post(u_acc, bias[i].reshape(1, D), res)
        else:
            x, ge = _tc_post_final(u_acc, bias[i].reshape(1, D), res)

    return (ge.reshape(2 * D), x[:N])


# speedup vs baseline: 1.2902x; 1.2902x over previous
import functools

import jax
import jax.numpy as jnp
from jax import lax
from jax.experimental import pallas as pl
from jax.experimental.pallas import tpu as pltpu
from jax.experimental.pallas import tpu_sc as plsc

N = 10000
F = 11
D = 128
H = 4
C = 32
L = 4

NP = 10016
RPS = NP // 16
W = 144
CH = 48
NSUB = 32


def _ceil_to(x, m):
    return (x + m - 1) // m * m


def _rowmask():
    rows = lax.broadcasted_iota(jnp.int32, (NP, 1), 0)
    return rows < N


def _embed_body(nf_ref, w_ref, b_ref, x_ref):
    x = jnp.dot(nf_ref[...], w_ref[...], preferred_element_type=jnp.float32)
    x = jnp.maximum(x + b_ref[...], 0.0)
    x_ref[...] = jnp.where(_rowmask(), x, 0.0)


def _tc_embed(nf_pad, W_emb_pad, b_emb):
    return pl.pallas_call(
        _embed_body,
        out_shape=jax.ShapeDtypeStruct((NP, D), jnp.float32),
    )(nf_pad, W_emb_pad, b_emb)


def _pre_body(x_ref, wl_ref, bl_ref, wr_ref, br_ref, a_ref, xl_ref, xre_ref):
    x = x_ref[...]
    mask = _rowmask()
    xl = jnp.dot(x, wl_ref[...], preferred_element_type=jnp.float32) + bl_ref[...]
    xr = jnp.dot(x, wr_ref[...], preferred_element_type=jnp.float32) + br_ref[...]
    xl = jnp.where(mask, xl, 0.0)
    xr = jnp.where(mask, xr, 0.0)
    s = xl + xr
    m = jnp.maximum(s, 0.2 * s)
    c = jnp.dot(m, a_ref[...], preferred_element_type=jnp.float32)
    xl_ref[...] = xl
    xre_ref[...] = jnp.concatenate(
        [xr, c, jnp.zeros((NP, W - D - H), jnp.float32)], axis=1)


def _tc_pre(x, Wl_i, bl_i, Wr_i, br_i, A_i):
    return pl.pallas_call(
        _pre_body,
        out_shape=(
            jax.ShapeDtypeStruct((NP, D), jnp.float32),
            jax.ShapeDtypeStruct((NP, W), jnp.float32),
        ),
    )(x, Wl_i, bl_i, Wr_i, br_i, A_i)


def _post_x(u_ref, bias_ref, res_ref):
    u = u_ref[0] + u_ref[1]
    den = jnp.concatenate(
        [jnp.broadcast_to(u[:, D + h:D + h + 1], (NP, C)) for h in range(H)],
        axis=1)
    out = u[:, :D] / (den + 1e-16)
    hv = jnp.maximum(out + bias_ref[...], 0.0)
    return jnp.where(_rowmask(), hv + res_ref[...], 0.0)


def _post_body(u_ref, bias_ref, res_ref, x_ref):
    x_ref[...] = _post_x(u_ref, bias_ref, res_ref)


def _tc_post(U, bias_i, res):
    return pl.pallas_call(
        _post_body,
        out_shape=jax.ShapeDtypeStruct((NP, D), jnp.float32),
    )(U, bias_i, res)


def _post_final_body(u_ref, bias_ref, res_ref, x_ref, ge_ref):
    x = _post_x(u_ref, bias_ref, res_ref)
    x_ref[...] = x
    gmean = jnp.sum(x, axis=0) * (1.0 / N)
    gmax = jnp.max(x, axis=0)
    ge_ref[...] = jnp.concatenate([gmean[None, :], gmax[None, :]], axis=0)


def _tc_post_final(U, bias_i, res):
    return pl.pallas_call(
        _post_final_body,
        out_shape=(
            jax.ShapeDtypeStruct((NP, D), jnp.float32),
            jax.ShapeDtypeStruct((2, D), jnp.float32),
        ),
    )(U, bias_i, res)


@functools.lru_cache(maxsize=None)
def _make_edge_pass(epad):
    k_chunks = epad // (NSUB * CH)
    assert k_chunks % 2 == 0
    mesh = plsc.VectorSubcoreMesh(core_axis_name="c", subcore_axis_name="s")

    nbuf = 2
    scratch = []
    for _ in range(nbuf):
        scratch += [
            pltpu.VMEM((CH,), jnp.int32),
            pltpu.VMEM((CH,), jnp.int32),
            pltpu.VMEM((CH,), jnp.int32),
            pltpu.VMEM((CH, D), jnp.float32),
            pltpu.VMEM((CH, W), jnp.float32),
            pltpu.VMEM((CH, W), jnp.float32),
            pltpu.SemaphoreType.DMA,
            pltpu.SemaphoreType.DMA,
            pltpu.SemaphoreType.DMA,
        ]
    scratch += [
        pltpu.VMEM((D,), jnp.float32),
        pltpu.VMEM_SHARED((NP, W), jnp.float32),
    ]

    @functools.partial(
        pl.kernel,
        mesh=mesh,
        compiler_params=pltpu.CompilerParams(
            use_tc_tiling_on_sc=False, needs_layout_passes=False),
        out_type=jax.ShapeDtypeStruct((2, NP, W), jnp.float32),
        scratch_types=scratch,
    )
    def edge_kernel(xl_hbm, xre_hbm, src_hbm, dst_hbm, att_hbm, out_hbm,
                    *refs):
        bufs = []
        for b in range(nbuf):
            (sidx, didx, didxs, xlv, xrv, msgv,
             semi, semg, sems) = refs[9 * b:9 * b + 9]
            bufs.append(dict(sidx=sidx, didx=didx, didxs=didxs, xlv=xlv,
                             xrv=xrv, msgv=msgv, semi=semi, semg=semg,
                             sems=sems))
        attv, usp = refs[9 * nbuf:]
        msgv = bufs[0]["msgv"]

        cid = lax.axis_index("c")
        sid = lax.axis_index("s")
        wid = sid * 2 + cid
        cbase = wid * k_chunks

        def _zrow(e, carry):
            for j in range(W // 16):
                msgv[e, pl.ds(16 * j, 16)] = jnp.zeros((16,), jnp.float32)
            return carry

        lax.fori_loop(0, CH, _zrow, 0)
        row0 = sid * RPS
        nfull = RPS // CH
        for t in range(nfull):
            pltpu.sync_copy(msgv, usp.at[pl.ds(row0 + t * CH, CH)])
        rem = RPS - nfull * CH
        if rem:
            pltpu.sync_copy(msgv.at[pl.ds(0, rem)],
                            usp.at[pl.ds(row0 + nfull * CH, rem)])
        pltpu.sync_copy(att_hbm, attv)
        plsc.subcore_barrier()

        atts = [attv[pl.ds(16 * j, 16)] for j in range(D // 16)]
        iot = jnp.arange(16, dtype=jnp.int32)

        def _fetch_idx(g, buf):
            base = (cbase + g) * CH
            pltpu.async_copy(src_hbm.at[pl.ds(base, CH)], buf["sidx"],
                             buf["semi"])
            pltpu.async_copy(dst_hbm.at[pl.ds(base, CH)], buf["didx"],
                             buf["semi"])

        def _wait_idx(buf):
            pltpu.make_async_copy(src_hbm.at[pl.ds(0, CH)], buf["sidx"],
                                  buf["semi"]).wait()
            pltpu.make_async_copy(dst_hbm.at[pl.ds(0, CH)], buf["didx"],
                                  buf["semi"]).wait()

        def _start_gathers(buf):
            pltpu.async_copy(xl_hbm.at[buf["sidx"]], buf["xlv"], buf["semg"])
            pltpu.async_copy(xre_hbm.at[buf["didx"]], buf["xrv"], buf["semg"])

        def _copy_didxs(buf):
            for j in range(0, CH, 16):
                j0 = min(j, CH - 16)
                buf["didxs"][pl.ds(j0, 16)] = buf["didx"][pl.ds(j0, 16)]

        def _drain_scatter(buf):
            pltpu.make_async_copy(buf["msgv"], usp.at[buf["didxs"]],
                                  buf["sems"]).wait()

        def _wait_gathers(buf):
            pltpu.make_async_copy(xl_hbm.at[buf["sidx"]], buf["xlv"],
                                  buf["semg"]).wait()
            pltpu.make_async_copy(xre_hbm.at[buf["didx"]], buf["xrv"],
                                  buf["semg"]).wait()

        def _compute_scatter(buf):
            xlv, xrv, msgv = buf["xlv"], buf["xrv"], buf["msgv"]

            @plsc.parallel_loop(0, CH, unroll=2)
            def _edge(e):
                xls = [xlv[e, pl.ds(16 * j, 16)] for j in range(D // 16)]
                cv = xrv[e, pl.ds(D, 16)]
                dvals = []
                for h in range(H):
                    acc = None
                    for jj in (2 * h, 2 * h + 1):
                        s = xls[jj] + xrv[e, pl.ds(16 * jj, 16)]
                        m = jnp.maximum(s, 0.2 * s)
                        am = m * atts[jj]
                        acc = am if acc is None else acc + am
                    dvals.append(jnp.sum(acc) - cv[h])
                tail = jnp.full((16,), 0.0, jnp.float32)
                for h in range(H):
                    bex = jnp.exp(jnp.full((16,), dvals[h], jnp.float32))
                    msgv[e, pl.ds(32 * h, 16)] = xls[2 * h] * bex
                    msgv[e, pl.ds(32 * h + 16, 16)] = xls[2 * h + 1] * bex
                    tail = jnp.where(iot == h, bex, tail)
                msgv[e, pl.ds(D, 16)] = tail

            pltpu.async_copy(msgv, usp.at[buf["didxs"]], buf["sems"],
                             add=True)

        pltpu.sync_copy(src_hbm.at[pl.ds(cbase * CH, CH)], bufs[0]["sidx"])
        pltpu.sync_copy(dst_hbm.at[pl.ds(cbase * CH, CH)], bufs[0]["didx"])
        _start_gathers(bufs[0])
        _fetch_idx(1, bufs[1])

        def _halfstep(g, cur, nxt):
            @pl.when(g + 1 < k_chunks)
            def _():
                _wait_idx(nxt)
                _start_gathers(nxt)

            _wait_gathers(cur)

            @pl.when(g >= 2)
            def _():
                _drain_scatter(cur)

            _copy_didxs(cur)

            @pl.when(g + 2 < k_chunks)
            def _():
                _fetch_idx(g + 2, cur)

            _compute_scatter(cur)

        def _pair(t, carry):
            _halfstep(2 * t, bufs[0], bufs[1])
            _halfstep(2 * t + 1, bufs[1], bufs[0])
            return carry

        lax.fori_loop(0, k_chunks // 2, _pair, 0)
        for b in bufs:
            _drain_scatter(b)
        plsc.subcore_barrier()

        for t in range(nfull):
            pltpu.sync_copy(usp.at[pl.ds(row0 + t * CH, CH)],
                            out_hbm.at[cid, pl.ds(row0 + t * CH, CH)])
        pltpu.sync_copy(usp.at[pl.ds(row0 + nfull * CH, rem)],
                        out_hbm.at[cid, pl.ds(row0 + nfull * CH, rem)])

    return edge_kernel


def kernel(node_features, edge_index, num_nodes, W_emb, b_emb,
           Wl, bl, Wr, br, att, bias):
    f32 = jnp.float32
    e_real = edge_index.shape[1] + N
    epad = _ceil_to(e_real, 2 * NSUB * CH)

    nf_pad = jnp.zeros((NP, 16), f32).at[:N, :F].set(node_features)
    wemb_pad = jnp.zeros((16, D), f32).at[:F, :].set(W_emb)
    loop = jnp.arange(N, dtype=jnp.int32)
    padfill = jnp.full((epad - e_real,), N, jnp.int32)
    src = jnp.concatenate([edge_index[0], loop, padfill])
    dst = jnp.concatenate([edge_index[1], loop, padfill])
    attf = att.reshape(L, H * C)
    onehot = jnp.repeat(jnp.eye(H, dtype=f32), C, axis=0)

    edge_pass = _make_edge_pass(epad)

    x = _tc_embed(nf_pad, wemb_pad, b_emb.reshape(1, D))
    ge = None
    for i in range(L):
        a_mat = attf[i][:, None] * onehot
        xl_tab, xre_tab = _tc_pre(x, Wl[i], bl[i].reshape(1, D),
                                  Wr[i], br[i].reshape(1, D), a_mat)
        u_acc = edge_pass(xl_tab, xre_tab, src, dst, attf[i])
        res = jnp.zeros((NP, D), f32) if i == 0 else x
        if i < L - 1:
            x = _tc_post(u_acc, bias[i].reshape(1, D), res)
        else:
            x, ge = _tc_post_final(u_acc, bias[i].reshape(1, D), res)

    return (ge.reshape(2 * D), x[:N])


# speedup vs baseline: 1.3992x; 1.0845x over previous
import functools

import jax
import jax.numpy as jnp
import numpy as np
from jax import lax
from jax.experimental import pallas as pl
from jax.experimental.pallas import tpu as pltpu
from jax.experimental.pallas import tpu_sc as plsc

N = 10000
F = 11
D = 128
H = 4
C = 32
L = 4

NP = 10016
RPS = NP // 16
W = 144
CH = 48
NSUB = 32


def _ceil_to(x, m):
    return (x + m - 1) // m * m


_PI = np.empty(D, dtype=np.int64)
for _h in range(H):
    for _c in range(C):
        _PI[C * _h + _c] = (_c // 4) * 16 + 4 * _h + (_c % 4)
_P2S = np.argsort(_PI)
_PM = np.zeros((D, D), dtype=np.float32)
_PM[_PI, np.arange(D)] = 1.0


def _rowmask():
    rows = lax.broadcasted_iota(jnp.int32, (NP, 1), 0)
    return rows < N


def _embed_body(nf_ref, w_ref, b_ref, x_ref):
    x = jnp.dot(nf_ref[...], w_ref[...], preferred_element_type=jnp.float32)
    x = jnp.maximum(x + b_ref[...], 0.0)
    x_ref[...] = jnp.where(_rowmask(), x, 0.0)


def _tc_embed(nf_pad, W_emb_pad, b_emb):
    return pl.pallas_call(
        _embed_body,
        out_shape=jax.ShapeDtypeStruct((NP, D), jnp.float32),
    )(nf_pad, W_emb_pad, b_emb)


def _pre_body(x_ref, wl_ref, bl_ref, wr_ref, br_ref, a_ref, xl_ref, xre_ref):
    x = x_ref[...]
    mask = _rowmask()
    xl = jnp.dot(x, wl_ref[...], preferred_element_type=jnp.float32) + bl_ref[...]
    xr = jnp.dot(x, wr_ref[...], preferred_element_type=jnp.float32) + br_ref[...]
    xl = jnp.where(mask, xl, 0.0)
    xr = jnp.where(mask, xr, 0.0)
    s = xl + xr
    m = jnp.maximum(s, 0.2 * s)
    c = jnp.dot(m, a_ref[...], preferred_element_type=jnp.float32)
    c4 = jnp.concatenate(
        [jnp.broadcast_to(c[:, h:h + 1], (NP, 4)) for h in range(H)], axis=1)
    xl_ref[...] = xl
    xre_ref[...] = jnp.concatenate([xr, c4], axis=1)


def _tc_pre(x, Wl_i, bl_i, Wr_i, br_i, A_i):
    return pl.pallas_call(
        _pre_body,
        out_shape=(
            jax.ShapeDtypeStruct((NP, D), jnp.float32),
            jax.ShapeDtypeStruct((NP, W), jnp.float32),
        ),
    )(x, Wl_i, bl_i, Wr_i, br_i, A_i)


_GB = 4
_BR = NP // _GB


def _post_x(u_ref, bias_ref, res_ref, pm_ref):
    i = pl.program_id(0)
    u = u_ref[0] + u_ref[1]
    den = jnp.concatenate([u[:, D:W]] * (D // 16), axis=1)
    out_p = u[:, :D] / (den + 1e-16)
    out = jnp.dot(out_p, pm_ref[...],
                  preferred_element_type=jnp.float32)
    hv = jnp.maximum(out + bias_ref[...], 0.0)
    rows = i * _BR + lax.broadcasted_iota(jnp.int32, (_BR, 1), 0)
    return jnp.where(rows < N, hv + res_ref[...], 0.0)


_POST_SPECS = dict(
    grid=(_GB,),
    in_specs=[
        pl.BlockSpec((2, _BR, W), lambda i: (0, i, 0)),
        pl.BlockSpec((1, D), lambda i: (0, 0)),
        pl.BlockSpec((_BR, D), lambda i: (i, 0)),
        pl.BlockSpec((D, D), lambda i: (0, 0)),
    ],
)


def _post_body(u_ref, bias_ref, res_ref, pm_ref, x_ref):
    x_ref[...] = _post_x(u_ref, bias_ref, res_ref, pm_ref)


def _tc_post(U, bias_i, res, Pm):
    return pl.pallas_call(
        _post_body,
        **_POST_SPECS,
        out_specs=pl.BlockSpec((_BR, D), lambda i: (i, 0)),
        out_shape=jax.ShapeDtypeStruct((NP, D), jnp.float32),
    )(U, bias_i, res, Pm)


def _post_final_body(u_ref, bias_ref, res_ref, pm_ref, x_ref, ge_ref):
    i = pl.program_id(0)
    x = _post_x(u_ref, bias_ref, res_ref, pm_ref)
    x_ref[...] = x
    gmean = jnp.sum(x, axis=0) * (1.0 / N)
    gmax = jnp.max(x, axis=0)
    part = jnp.concatenate([gmean[None, :], gmax[None, :]], axis=0)

    @pl.when(i == 0)
    def _():
        ge_ref[...] = jnp.zeros((2, D), jnp.float32)

    ge_ref[0:1, :] += part[0:1, :]
    ge_ref[1:2, :] = jnp.maximum(ge_ref[1:2, :], part[1:2, :])


def _tc_post_final(U, bias_i, res, Pm):
    return pl.pallas_call(
        _post_final_body,
        **_POST_SPECS,
        out_specs=(
            pl.BlockSpec((_BR, D), lambda i: (i, 0)),
            pl.BlockSpec((2, D), lambda i: (0, 0)),
        ),
        out_shape=(
            jax.ShapeDtypeStruct((NP, D), jnp.float32),
            jax.ShapeDtypeStruct((2, D), jnp.float32),
        ),
    )(U, bias_i, res, Pm)


@functools.lru_cache(maxsize=None)
def _make_edge_pass(epad):
    k_chunks = epad // (NSUB * CH)
    assert k_chunks % 2 == 0
    mesh = plsc.VectorSubcoreMesh(core_axis_name="c", subcore_axis_name="s")

    nbuf = 2
    scratch = []
    for _ in range(nbuf):
        scratch += [
            pltpu.VMEM((CH,), jnp.int32),
            pltpu.VMEM((CH,), jnp.int32),
            pltpu.VMEM((CH,), jnp.int32),
            pltpu.VMEM((CH, D), jnp.float32),
            pltpu.VMEM((CH, W), jnp.float32),
            pltpu.VMEM((CH, W), jnp.float32),
            pltpu.SemaphoreType.DMA,
            pltpu.SemaphoreType.DMA,
            pltpu.SemaphoreType.DMA,
        ]
    scratch += [
        pltpu.VMEM((D,), jnp.float32),
        pltpu.VMEM_SHARED((NP, W), jnp.float32),
    ]

    @functools.partial(
        pl.kernel,
        mesh=mesh,
        compiler_params=pltpu.CompilerParams(
            use_tc_tiling_on_sc=False, needs_layout_passes=False),
        out_type=jax.ShapeDtypeStruct((2, NP, W), jnp.float32),
        scratch_types=scratch,
    )
    def edge_kernel(xl_hbm, xre_hbm, src_hbm, dst_hbm, att_hbm, out_hbm,
                    *refs):
        bufs = []
        for b in range(nbuf):
            (sidx, didx, didxs, xlv, xrv, msgv,
             semi, semg, sems) = refs[9 * b:9 * b + 9]
            bufs.append(dict(sidx=sidx, didx=didx, didxs=didxs, xlv=xlv,
                             xrv=xrv, msgv=msgv, semi=semi, semg=semg,
                             sems=sems))
        attv, usp = refs[9 * nbuf:]
        msgv = bufs[0]["msgv"]

        cid = lax.axis_index("c")
        sid = lax.axis_index("s")
        wid = sid * 2 + cid
        cbase = wid * k_chunks

        def _zrow(e, carry):
            for j in range(W // 16):
                msgv[e, pl.ds(16 * j, 16)] = jnp.zeros((16,), jnp.float32)
            return carry

        lax.fori_loop(0, CH, _zrow, 0)
        row0 = sid * RPS
        nfull = RPS // CH
        for t in range(nfull):
            pltpu.sync_copy(msgv, usp.at[pl.ds(row0 + t * CH, CH)])
        rem = RPS - nfull * CH
        if rem:
            pltpu.sync_copy(msgv.at[pl.ds(0, rem)],
                            usp.at[pl.ds(row0 + nfull * CH, rem)])
        pltpu.sync_copy(att_hbm, attv)
        plsc.subcore_barrier()

        atts = [attv[pl.ds(16 * j, 16)] for j in range(D // 16)]
        lane = jnp.arange(16, dtype=jnp.int32)
        quad = (lane >> 2) << 2
        hi_idx = quad + 3
        lo_idx = quad

        def _fetch_idx(g, buf):
            base = (cbase + g) * CH
            pltpu.async_copy(src_hbm.at[pl.ds(base, CH)], buf["sidx"],
                             buf["semi"])
            pltpu.async_copy(dst_hbm.at[pl.ds(base, CH)], buf["didx"],
                             buf["semi"])

        def _wait_idx(buf):
            pltpu.make_async_copy(src_hbm.at[pl.ds(0, CH)], buf["sidx"],
                                  buf["semi"]).wait()
            pltpu.make_async_copy(dst_hbm.at[pl.ds(0, CH)], buf["didx"],
                                  buf["semi"]).wait()

        def _start_gathers(buf):
            pltpu.async_copy(xl_hbm.at[buf["sidx"]], buf["xlv"], buf["semg"])
            pltpu.async_copy(xre_hbm.at[buf["didx"]], buf["xrv"], buf["semg"])

        def _copy_didxs(buf):
            for j in range(0, CH, 16):
                j0 = min(j, CH - 16)
                buf["didxs"][pl.ds(j0, 16)] = buf["didx"][pl.ds(j0, 16)]

        def _drain_scatter(buf):
            pltpu.make_async_copy(buf["msgv"], usp.at[buf["didxs"]],
                                  buf["sems"]).wait()

        def _wait_gathers(buf):
            pltpu.make_async_copy(xl_hbm.at[buf["sidx"]], buf["xlv"],
                                  buf["semg"]).wait()
            pltpu.make_async_copy(xre_hbm.at[buf["didx"]], buf["xrv"],
                                  buf["semg"]).wait()

        def _compute_scatter(buf):
            xlv, xrv, msgv = buf["xlv"], buf["xrv"], buf["msgv"]

            @plsc.parallel_loop(0, CH, unroll=2)
            def _edge(e):
                xls = [xlv[e, pl.ds(16 * j, 16)] for j in range(D // 16)]
                cv = xrv[e, pl.ds(D, 16)]
                p = None
                for j in range(D // 16):
                    s = xls[j] + xrv[e, pl.ds(16 * j, 16)]
                    m = jnp.maximum(s, 0.2 * s)
                    am = m * atts[j]
                    p = am if p is None else p + am
                cum = jnp.cumsum(p)
                excl = cum - p
                sv = cum[hi_idx] - excl[lo_idx]
                bexp = jnp.exp(sv - cv)
                for j in range(D // 16):
                    msgv[e, pl.ds(16 * j, 16)] = xls[j] * bexp
                msgv[e, pl.ds(D, 16)] = bexp

            pltpu.async_copy(msgv, usp.at[buf["didxs"]], buf["sems"],
                             add=True)

        pltpu.sync_copy(src_hbm.at[pl.ds(cbase * CH, CH)], bufs[0]["sidx"])
        pltpu.sync_copy(dst_hbm.at[pl.ds(cbase * CH, CH)], bufs[0]["didx"])
        _start_gathers(bufs[0])
        _fetch_idx(1, bufs[1])

        def _halfstep(g, cur, nxt):
            @pl.when(g + 1 < k_chunks)
            def _():
                _wait_idx(nxt)
                _start_gathers(nxt)

            _wait_gathers(cur)

            @pl.when(g >= 2)
            def _():
                _drain_scatter(cur)

            _copy_didxs(cur)

            @pl.when(g + 2 < k_chunks)
            def _():
                _fetch_idx(g + 2, cur)

            _compute_scatter(cur)

        def _pair(t, carry):
            _halfstep(2 * t, bufs[0], bufs[1])
            _halfstep(2 * t + 1, bufs[1], bufs[0])
            return carry

        lax.fori_loop(0, k_chunks // 2, _pair, 0)
        for b in bufs:
            _drain_scatter(b)
        plsc.subcore_barrier()

        for t in range(nfull):
            pltpu.sync_copy(usp.at[pl.ds(row0 + t * CH, CH)],
                            out_hbm.at[cid, pl.ds(row0 + t * CH, CH)])
        pltpu.sync_copy(usp.at[pl.ds(row0 + nfull * CH, rem)],
                        out_hbm.at[cid, pl.ds(row0 + nfull * CH, rem)])

    return edge_kernel


def kernel(node_features, edge_index, num_nodes, W_emb, b_emb,
           Wl, bl, Wr, br, att, bias):
    f32 = jnp.float32
    e_real = edge_index.shape[1] + N
    epad = _ceil_to(e_real, 2 * NSUB * CH)

    nf_pad = jnp.zeros((NP, 16), f32).at[:N, :F].set(node_features)
    wemb_pad = jnp.zeros((16, D), f32).at[:F, :].set(W_emb)
    loop = jnp.arange(N, dtype=jnp.int32)
    padfill = jnp.full((epad - e_real,), N, jnp.int32)
    src = jnp.concatenate([edge_index[0], loop, padfill])
    dst = jnp.concatenate([edge_index[1], loop, padfill])
    attf = att.reshape(L, H * C)
    onehot = jnp.repeat(jnp.eye(H, dtype=f32), C, axis=0)
    p2s = jnp.asarray(_P2S)
    pm = jnp.asarray(_PM)

    edge_pass = _make_edge_pass(epad)

    x = _tc_embed(nf_pad, wemb_pad, b_emb.reshape(1, D))
    ge = None
    for i in range(L):
        attp = attf[i][p2s]
        a_mat = (attf[i][:, None] * onehot)[p2s]
        xl_tab, xre_tab = _tc_pre(x, Wl[i][:, p2s], bl[i][p2s].reshape(1, D),
                                  Wr[i][:, p2s], br[i][p2s].reshape(1, D),
                                  a_mat)
        u_acc = edge_pass(xl_tab, xre_tab, src, dst, attp)
        res = jnp.zeros((NP, D), f32) if i == 0 else x
        if i < L - 1:
            x = _tc_post(u_acc, bias[i].reshape(1, D), res, pm)
        else:
            x, ge = _tc_post_final(u_acc, bias[i].reshape(1, D), res, pm)

    return (ge.reshape(2 * D), x[:N])
